# baseline (speedup 1.0000x reference)
import functools

import jax
import jax.numpy as jnp
from jax import lax
from jax.experimental import pallas as pl
from jax.experimental.pallas import tpu as pltpu
from jax.experimental.pallas import tpu_sc as plsc

_V = 1024
_D = 128
_CHUNK = 128


@functools.cache
def _build(n_total: int, nw: int, nchunk: int):
    mesh = plsc.VectorSubcoreMesh(core_axis_name="c", subcore_axis_name="s")

    @functools.partial(
        pl.kernel,
        mesh=mesh,
        out_type=jax.ShapeDtypeStruct((n_total, _D), jnp.float32),
        scratch_types=[
            pltpu.VMEM((nchunk, _CHUNK), jnp.int32),
            pltpu.VMEM((2, _CHUNK, _D), jnp.float32),
            pltpu.SemaphoreType.DMA,
            pltpu.SemaphoreType.DMA,
        ],
    )
    def gather_kernel(ids_hbm, cb_hbm, out_hbm, idx_v, rows_v, gsem, osem):
        cid = lax.axis_index("c")
        sid = lax.axis_index("s")
        nc = nw // 16
        wid = sid * nc + cid

        pltpu.sync_copy(ids_hbm.at[wid], idx_v)

        base = wid * (nchunk * _CHUNK)

        def gather_chunk(j, b):
            return pltpu.async_copy(cb_hbm.at[idx_v.at[j]], rows_v.at[b], gsem)

        def write_desc(j, b):
            return pltpu.make_async_copy(
                rows_v.at[b], out_hbm.at[pl.ds(base + j * _CHUNK, _CHUNK)], osem)

        gather_chunk(0, 0)

        def body(g, carry):
            for b in range(2):
                j = 2 * g + b
                pltpu.make_async_copy(
                    cb_hbm.at[idx_v.at[j]], rows_v.at[b], gsem).wait()
                @pl.when(j >= 1)
                def _():
                    write_desc(j - 1, 1 - b).wait()
                @pl.when(j + 1 < nchunk)
                def _():
                    gather_chunk(j + 1, 1 - b)
                write_desc(j, b).start()
            return carry

        lax.fori_loop(0, nchunk // 2, body, 0)
        write_desc(nchunk - 1, (nchunk - 1) % 2).wait()

    return gather_kernel


def kernel(code_ids, codebook):
    b, k = code_ids.shape
    n = b * k
    info = plsc.get_sparse_core_info()
    nw = info.num_cores * info.num_subcores
    per_w = n // nw
    assert n % nw == 0 and per_w % _CHUNK == 0, (n, nw)
    nchunk = per_w // _CHUNK
    ids = code_ids.reshape(nw, nchunk, _CHUNK).astype(jnp.int32)
    out = _build(n, nw, nchunk)(ids, codebook)
    return out.reshape(b, k, _D)


# speedup vs baseline: 1.3479x; 1.3479x over previous
import functools

import jax
import jax.numpy as jnp
from jax import lax
from jax.experimental import pallas as pl
from jax.experimental.pallas import tpu as pltpu
from jax.experimental.pallas import tpu_sc as plsc

_V = 1024
_D = 128
_CHUNK = 128


@functools.cache
def _build(n_total: int, nw: int, nchunk: int):
    mesh = plsc.VectorSubcoreMesh(core_axis_name="c", subcore_axis_name="s")

    @functools.partial(
        pl.kernel,
        mesh=mesh,
        out_type=jax.ShapeDtypeStruct((n_total, _D), jnp.float32),
        scratch_types=[
            pltpu.VMEM((nchunk, _CHUNK), jnp.int32),
            pltpu.VMEM((4, _CHUNK, _D), jnp.float32),
            pltpu.VMEM_SHARED((_V, _D), jnp.float32),
            pltpu.SemaphoreType.DMA,
            pltpu.SemaphoreType.DMA,
        ],
    )
    def gather_kernel(ids_hbm, cb_hbm, out_hbm, idx_v, rows_v, cb_sh, gsem, osem):
        cid = lax.axis_index("c")
        sid = lax.axis_index("s")
        nc = nw // 16
        wid = sid * nc + cid

        @pl.when(sid == 0)
        def _():
            pltpu.sync_copy(cb_hbm, cb_sh)

        plsc.subcore_barrier()

        pltpu.sync_copy(ids_hbm.at[wid], idx_v)

        base = wid * (nchunk * _CHUNK)
        nbuf = 4

        def gather_chunk(j, b):
            return pltpu.async_copy(cb_sh.at[idx_v.at[j]], rows_v.at[b], gsem)

        def write_desc(j, b):
            return pltpu.make_async_copy(
                rows_v.at[b], out_hbm.at[pl.ds(base + j * _CHUNK, _CHUNK)], osem)

        for b in range(nbuf - 1):
            gather_chunk(b, b)

        def body(g, carry):
            for b in range(nbuf):
                j = nbuf * g + b
                pltpu.make_async_copy(
                    cb_sh.at[idx_v.at[j]], rows_v.at[b], gsem).wait()
                @pl.when(j >= 1)
                def _():
                    write_desc(j - 1, (b + nbuf - 1) % nbuf).wait()
                @pl.when(j + nbuf - 1 < nchunk)
                def _():
                    gather_chunk(j + nbuf - 1, (b + nbuf - 1) % nbuf)
                write_desc(j, b).start()
            return carry

        lax.fori_loop(0, nchunk // nbuf, body, 0)
        write_desc(nchunk - 1, (nchunk - 1) % nbuf).wait()

    return gather_kernel


def kernel(code_ids, codebook):
    b, k = code_ids.shape
    n = b * k
    info = plsc.get_sparse_core_info()
    nw = info.num_cores * info.num_subcores
    per_w = n // nw
    assert n % nw == 0 and per_w % _CHUNK == 0, (n, nw)
    nchunk = per_w // _CHUNK
    ids = code_ids.reshape(nw, nchunk, _CHUNK).astype(jnp.int32)
    out = _build(n, nw, nchunk)(ids, codebook)
    return out.reshape(b, k, _D)
